# Initial kernel scaffold; baseline (speedup 1.0000x reference)
import jax
import jax.numpy as jnp
from jax.experimental import pallas as pl


def kernel(predictions, targets, weight_tensor):
    raise NotImplementedError("write your pallas kernel here")


# speedup vs baseline: 4.0733x; 4.0733x over previous
import functools

import jax
import jax.numpy as jnp
from jax import lax
from jax.experimental import pallas as pl
from jax.experimental.pallas import tpu as pltpu
from jax.experimental.pallas import tpu_sc as plsc

_MIN_RATING = 0.0
_SCALE = 100.0
_NUM_BINS = 1001
_TBL_PAD = 1008
_BATCH = 16384
_NC, _NS, _L = 2, 16, 16
_NW = _NC * _NS
_BPW = _BATCH // _NW
_VECS = _BPW // _L


def _body(pred_hbm, targ_hbm, w_hbm, out_hbm, w_v, pred_v, targ_v, part_v):
    c = lax.axis_index("c")
    s = lax.axis_index("s")
    wid = s * _NC + c
    base = wid * _BPW
    pltpu.sync_copy(w_hbm, w_v)
    pltpu.sync_copy(pred_hbm.at[pl.ds(base, _BPW)], pred_v)
    pltpu.sync_copy(targ_hbm.at[pl.ds(base, _BPW)], targ_v)
    acc = jnp.zeros((_L,), jnp.float32)
    for i in range(_VECS):
        p = pred_v[pl.ds(i * _L, _L)]
        t = targ_v[pl.ds(i * _L, _L)]
        idx = ((t - _MIN_RATING) * _SCALE + 0.5).astype(jnp.int32)
        idx = jnp.minimum(jnp.maximum(idx, 0), _NUM_BINS - 1)
        w = plsc.load_gather(w_v, [idx])
        d = p - t
        acc = acc + w * d * d
    part_v[...] = acc
    pltpu.sync_copy(part_v, out_hbm.at[wid])


@functools.partial(jax.jit, static_argnames=())
def kernel(predictions, targets, weight_tensor):
    wt = jnp.zeros((_TBL_PAD,), jnp.float32).at[:_NUM_BINS].set(weight_tensor)
    mesh = plsc.VectorSubcoreMesh(core_axis_name="c", subcore_axis_name="s")
    partials = pl.kernel(
        _body,
        out_type=jax.ShapeDtypeStruct((_NW, _L), jnp.float32),
        mesh=mesh,
        scratch_types=[
            pltpu.VMEM((_TBL_PAD,), jnp.float32),
            pltpu.VMEM((_BPW,), jnp.float32),
            pltpu.VMEM((_BPW,), jnp.float32),
            pltpu.VMEM((_L,), jnp.float32),
        ],
        compiler_params=pltpu.CompilerParams(needs_layout_passes=False),
    )(predictions, targets, wt)
    return jnp.sum(partials) / _BATCH


# speedup vs baseline: 4.9295x; 1.2102x over previous
import functools

import jax
import jax.numpy as jnp
from jax import lax
from jax.experimental import pallas as pl
from jax.experimental.pallas import tpu as pltpu
from jax.experimental.pallas import tpu_sc as plsc

_MIN_RATING = 0.0
_SCALE = 100.0
_NUM_BINS = 1001
_BATCH = 16384
_NS, _L = 16, 16
_BPW = _BATCH // _NS
_VECS = _BPW // _L


def _body(pred_hbm, targ_hbm, w_hbm, out_hbm,
          w_v, pred_v, targ_v, part_v, red_v, shared, sem):
    sid = lax.axis_index("s")
    base = sid * _BPW
    cw = pltpu.async_copy(w_hbm, w_v, sem)
    cp = pltpu.async_copy(pred_hbm.at[pl.ds(base, _BPW)], pred_v, sem)
    ct = pltpu.async_copy(targ_hbm.at[pl.ds(base, _BPW)], targ_v, sem)
    cw.wait()
    cp.wait()
    ct.wait()
    acc = jnp.zeros((_L,), jnp.float32)
    for i in range(_VECS):
        p = pred_v[pl.ds(i * _L, _L)]
        t = targ_v[pl.ds(i * _L, _L)]
        idx = ((t - _MIN_RATING) * _SCALE + 0.5).astype(jnp.int32)
        idx = jnp.minimum(jnp.maximum(idx, 0), _NUM_BINS - 1)
        w = plsc.load_gather(w_v, [idx])
        d = p - t
        acc = acc + w * d * d
    part_v[...] = acc
    pltpu.sync_copy(part_v, shared.at[pl.ds(sid * _L, _L)])
    plsc.subcore_barrier()

    @pl.when(sid == 0)
    def _():
        pltpu.sync_copy(shared, red_v)
        tot = red_v[pl.ds(0, _L)]
        for i in range(1, _NS):
            tot = tot + red_v[pl.ds(i * _L, _L)]
        mean = lax.reduce_sum_p.bind(tot * (1.0 / _BATCH), axes=(0,))
        part_v[...] = jnp.full((_L,), 0.0, jnp.float32) + mean
        pltpu.sync_copy(part_v.at[pl.ds(0, 8)], out_hbm)


@functools.partial(jax.jit, static_argnames=())
def kernel(predictions, targets, weight_tensor):
    mesh = plsc.VectorSubcoreMesh(
        core_axis_name="c", subcore_axis_name="s", num_cores=1)
    out = pl.kernel(
        _body,
        out_type=jax.ShapeDtypeStruct((8,), jnp.float32),
        mesh=mesh,
        scratch_types=[
            pltpu.VMEM((_NUM_BINS,), jnp.float32),
            pltpu.VMEM((_BPW,), jnp.float32),
            pltpu.VMEM((_BPW,), jnp.float32),
            pltpu.VMEM((_L,), jnp.float32),
            pltpu.VMEM((_NS * _L,), jnp.float32),
            pltpu.VMEM_SHARED((_NS * _L,), jnp.float32),
            pltpu.SemaphoreType.DMA,
        ],
        compiler_params=pltpu.CompilerParams(needs_layout_passes=False),
    )(predictions, targets, weight_tensor)
    return out[0]


# speedup vs baseline: 5.0786x; 1.0302x over previous
import functools

import jax
import jax.numpy as jnp
from jax import lax
from jax.experimental import pallas as pl
from jax.experimental.pallas import tpu as pltpu
from jax.experimental.pallas import tpu_sc as plsc

_MIN_RATING = 0.0
_SCALE = 100.0
_NUM_BINS = 1001
_BATCH = 16384
_NS, _L = 16, 16
_BPW = _BATCH // _NS
_VECS = _BPW // _L


def _body(pred_hbm, targ_hbm, w_hbm, out_hbm,
          w_v, pred_v, targ_v, part_v, red_v, shared, sem):
    sid = lax.axis_index("s")
    base = sid * _BPW
    cw = pltpu.async_copy(w_hbm, w_v, sem)
    cp = pltpu.async_copy(pred_hbm.at[pl.ds(base, _BPW)], pred_v, sem)
    ct = pltpu.async_copy(targ_hbm.at[pl.ds(base, _BPW)], targ_v, sem)
    cw.wait()
    cp.wait()
    ct.wait()
    def _step(i, acc):
        p = pred_v[pl.ds(i * _L, _L)]
        t = targ_v[pl.ds(i * _L, _L)]
        idx = ((t - _MIN_RATING) * _SCALE + 0.5).astype(jnp.int32)
        idx = jnp.minimum(jnp.maximum(idx, 0), _NUM_BINS - 1)
        w = plsc.load_gather(w_v, [idx])
        d = p - t
        return acc + w * d * d
    acc = lax.fori_loop(0, _VECS, _step, jnp.zeros((_L,), jnp.float32))
    part_v[...] = acc
    pltpu.sync_copy(part_v, shared.at[pl.ds(sid * _L, _L)])
    plsc.subcore_barrier()

    @pl.when(sid == 0)
    def _():
        pltpu.sync_copy(shared, red_v)
        tot = red_v[pl.ds(0, _L)]
        for i in range(1, _NS):
            tot = tot + red_v[pl.ds(i * _L, _L)]
        mean = lax.reduce_sum_p.bind(tot * (1.0 / _BATCH), axes=(0,))
        part_v[...] = jnp.full((_L,), 0.0, jnp.float32) + mean
        pltpu.sync_copy(part_v.at[pl.ds(0, 8)], out_hbm)


@functools.partial(jax.jit, static_argnames=())
def kernel(predictions, targets, weight_tensor):
    mesh = plsc.VectorSubcoreMesh(
        core_axis_name="c", subcore_axis_name="s", num_cores=1)
    out = pl.kernel(
        _body,
        out_type=jax.ShapeDtypeStruct((8,), jnp.float32),
        mesh=mesh,
        scratch_types=[
            pltpu.VMEM((_NUM_BINS,), jnp.float32),
            pltpu.VMEM((_BPW,), jnp.float32),
            pltpu.VMEM((_BPW,), jnp.float32),
            pltpu.VMEM((_L,), jnp.float32),
            pltpu.VMEM((_NS * _L,), jnp.float32),
            pltpu.VMEM_SHARED((_NS * _L,), jnp.float32),
            pltpu.SemaphoreType.DMA,
        ],
        compiler_params=pltpu.CompilerParams(needs_layout_passes=False),
    )(predictions, targets, weight_tensor)
    return out[0]
